# baseline (speedup 1.0000x reference)
import functools

import jax
import jax.numpy as jnp
from jax import lax
from jax.experimental import pallas as pl
from jax.experimental.pallas import tpu as pltpu
from jax.experimental.pallas import tpu_sc as plsc

N = 10000
E = 320000
D = 128
BN_EPS = 1e-5

S = 1
EPS = E // S
NC, NS, L = 2, 16, 16
NW = NC * NS
EPW = EPS // NW
B = 40
C = EPW // B
NP = 10240
RPT = NP // NS


_BE = 3200


def _rne_bf16_bits(w):
    return w + jnp.int32(0x7FFF) + ((w >> 16) & jnp.int32(1))


def _proj_body(a_ref, w_ref, b_ref, o_ref):
    y = (
        jnp.dot(a_ref[...], w_ref[...], preferred_element_type=jnp.float32)
        + b_ref[...]
    )
    w1 = _rne_bf16_bits(lax.bitcast_convert_type(y[:, : D // 2], jnp.int32))
    w2 = _rne_bf16_bits(lax.bitcast_convert_type(y[:, D // 2 :], jnp.int32))
    o_ref[...] = ((w1 >> 16) & jnp.int32(0xFFFF)) | (w2 & jnp.int32(-65536))


def _xpack_body(x_ref, o_ref):
    y = x_ref[...]
    w1 = _rne_bf16_bits(lax.bitcast_convert_type(y[:, : D // 2], jnp.int32))
    w2 = _rne_bf16_bits(lax.bitcast_convert_type(y[:, D // 2 :], jnp.int32))
    o_ref[...] = ((w1 >> 16) & jnp.int32(0xFFFF)) | (w2 & jnp.int32(-65536))


_xpack = pl.pallas_call(
    _xpack_body,
    out_shape=jax.ShapeDtypeStruct((N, D // 2), jnp.int32),
)


def _make_proj(s):
    nb = EPS // _BE
    return pl.pallas_call(
        _proj_body,
        grid=(nb,),
        in_specs=[
            pl.BlockSpec((_BE, D), lambda i: (i + s * nb, 0)),
            pl.BlockSpec((D, D), lambda i: (0, 0)),
            pl.BlockSpec((1, D), lambda i: (0, 0)),
        ],
        out_specs=pl.BlockSpec((_BE, D // 2), lambda i: (i, 0)),
        out_shape=jax.ShapeDtypeStruct((EPS, D // 2), jnp.int32),
    )


def _sc_body(s, x_hbm, e_hbm, src_hbm, dst_hbm, zeros_hbm, out_hbm,
             sidx, didx, xg_v, em_v, msg_v, agg_sh,
             gsem0, gsem1, esem0, esem1, ss0, ss1, ds0, ds1, cs0, cs1):
    cid = lax.axis_index("c")
    sid = lax.axis_index("s")
    wid = cid * NS + sid
    gsems = (gsem0, gsem1)
    esems = (esem0, esem1)
    ssems = (ss0, ss1)
    dsems = (ds0, ds1)
    csems = (cs0, cs1)

    pltpu.sync_copy(zeros_hbm.at[pl.ds(sid * RPT, RPT)],
                    agg_sh.at[pl.ds(sid * RPT, RPT)])
    plsc.subcore_barrier()

    ibase = s * EPS + wid * EPW
    ebase = wid * EPW

    def sidx_desc(ci, b):
        return pltpu.make_async_copy(src_hbm.at[pl.ds(ibase + ci * B, B)],
                                     sidx.at[b], ssems[b])

    def didx_desc(ci, b):
        return pltpu.make_async_copy(dst_hbm.at[pl.ds(ibase + ci * B, B)],
                                     didx.at[b], dsems[b])

    def scat_desc(b):
        return pltpu.make_async_copy(msg_v.at[b], agg_sh.at[didx.at[b]],
                                     csems[b])

    def gather_desc(ci, b):
        return (
            pltpu.make_async_copy(x_hbm.at[sidx.at[b]], xg_v.at[b], gsems[b]),
            pltpu.make_async_copy(e_hbm.at[pl.ds(ebase + ci * B, B)],
                                  em_v.at[b], esems[b]),
        )

    def start(descs):
        for d_ in descs:
            d_.start()

    def wait(descs):
        for d_ in descs:
            d_.wait()

    def chunk_step(ci, b):
        nb = 1 - b

        @pl.when(ci + 1 < C)
        def _():
            sidx_desc(ci + 1, nb).wait()
            start(gather_desc(ci + 1, nb))

        @pl.when(ci >= 2)
        def _():
            scat_desc(b).wait()

        wait(gather_desc(ci, b))

        @pl.when(ci + 2 < C)
        def _():
            sidx_desc(ci + 2, b).start()
            didx_desc(ci + 2, b).start()

        def rows(ri, c2):
            m = jnp.int32(-65536)
            for dr in range(4):
                r = 4 * ri + dr
                for k in range(D // (2 * L)):
                    ks = pl.ds(k * L, L)
                    ew = em_v[b, r, ks]
                    xw = xg_v[b, r, ks]
                    lo = (lax.bitcast_convert_type(xw << 16, jnp.float32)
                          + lax.bitcast_convert_type(ew << 16, jnp.float32))
                    hi = (lax.bitcast_convert_type(xw & m, jnp.float32)
                          + lax.bitcast_convert_type(ew & m, jnp.float32))
                    msg_v[b, r, ks] = jnp.maximum(lo, 0.0)
                    msg_v[b, r, pl.ds(D // 2 + k * L, L)] = jnp.maximum(hi, 0.0)
            return c2

        lax.fori_loop(0, B // 4, rows, 0)
        didx_desc(ci, b).wait()
        pltpu.async_copy(msg_v.at[b], agg_sh.at[didx.at[b]], csems[b],
                         add=True)

    sidx_desc(0, 0).start()
    didx_desc(0, 0).start()
    sidx_desc(0, 0).wait()
    start(gather_desc(0, 0))
    sidx_desc(1, 1).start()
    didx_desc(1, 1).start()

    def pair(g, carry):
        chunk_step(2 * g, 0)
        chunk_step(2 * g + 1, 1)
        return carry

    lax.fori_loop(0, C // 2, pair, 0)
    if C % 2:
        chunk_step(C - 1, 0)
    scat_desc((C - 2) % 2).wait()
    scat_desc((C - 1) % 2).wait()
    plsc.subcore_barrier()

    pltpu.sync_copy(agg_sh.at[pl.ds(sid * RPT, RPT)],
                    out_hbm.at[cid, pl.ds(sid * RPT, RPT)])


def _make_sc(s):
    return pl.kernel(
        functools.partial(_sc_body, s),
        out_type=jax.ShapeDtypeStruct((NC, NP, D), jnp.float32),
        mesh=plsc.VectorSubcoreMesh(core_axis_name="c", subcore_axis_name="s"),
        scratch_types=[
            pltpu.VMEM((2, B), jnp.int32),
            pltpu.VMEM((2, B), jnp.int32),
            pltpu.VMEM((2, B, D // 2), jnp.int32),
            pltpu.VMEM((2, B, D // 2), jnp.int32),
            pltpu.VMEM((2, B, D), jnp.float32),
            pltpu.VMEM_SHARED((NP, D), jnp.float32),
        ] + [pltpu.SemaphoreType.DMA] * 10,
        compiler_params=pltpu.CompilerParams(use_tc_tiling_on_sc=False),
    )


def _mlp_body(x_ref, p_ref, w1_ref, b1_ref, g_ref, bt_ref, w2_ref,
              b2_ref, eps_ref, o_ref):
    xv = x_ref[...]
    agg = p_ref[0, :N, :] + p_ref[1, :N, :]
    h = (1.0 + eps_ref[...]) * xv + agg
    h1 = jnp.dot(h, w1_ref[...], preferred_element_type=jnp.float32) + b1_ref[...]
    mean = jnp.mean(h1, axis=0, keepdims=True)
    ctr = h1 - mean
    var = jnp.mean(ctr * ctr, axis=0, keepdims=True)
    hn = ctr * lax.rsqrt(var + BN_EPS) * g_ref[...] + bt_ref[...]
    h2 = jnp.maximum(hn, 0.0)
    o_ref[...] = (
        xv + jnp.dot(h2, w2_ref[...], preferred_element_type=jnp.float32)
        + b2_ref[...]
    )


_mlp = pl.pallas_call(
    _mlp_body,
    out_shape=jax.ShapeDtypeStruct((N, D), jnp.float32),
)

_projs = [_make_proj(s) for s in range(S)]
_scs = [_make_sc(s) for s in range(S)]


def kernel(x, edge_index, edge_attr, We, be, W1, b1, gamma, beta, W2, b2, eps):
    src = edge_index[0]
    dst = edge_index[1]
    WeT = We.T
    be1 = be.reshape(1, D)
    zeros = jnp.zeros((NP, D), jnp.float32)
    xp = _xpack(x)
    e0 = _projs[0](edge_attr, WeT, be1)
    parts = _scs[0](xp, e0, src, dst, zeros)
    out = _mlp(x, parts, W1.T, b1.reshape(1, D),
               gamma.reshape(1, D), beta.reshape(1, D), W2.T,
               b2.reshape(1, D), eps.reshape(1, 1))
    return (out, edge_attr)


# speedup vs baseline: 1.4275x; 1.4275x over previous
import functools

import jax
import jax.numpy as jnp
from jax import lax
from jax.experimental import pallas as pl
from jax.experimental.pallas import tpu as pltpu
from jax.experimental.pallas import tpu_sc as plsc

N = 10000
E = 320000
D = 128
BN_EPS = 1e-5

S = 1
EPS = E // S
NC, NS, L = 2, 16, 16
NW = NC * NS
EPW = EPS // NW
B = 40
C = EPW // B
NP = 10240
RPT = NP // NS


_BE = 3200


def _rne_bf16_bits(w):
    return w + jnp.int32(0x7FFF) + ((w >> 16) & jnp.int32(1))


def _proj_body(a_ref, w_ref, b_ref, o_ref):
    y = (
        jnp.dot(a_ref[...], w_ref[...], preferred_element_type=jnp.float32)
        + b_ref[...]
    )
    w1 = _rne_bf16_bits(lax.bitcast_convert_type(y[:, : D // 2], jnp.int32))
    w2 = _rne_bf16_bits(lax.bitcast_convert_type(y[:, D // 2 :], jnp.int32))
    o_ref[...] = ((w1 >> 16) & jnp.int32(0xFFFF)) | (w2 & jnp.int32(-65536))


def _make_proj(s):
    nb = EPS // _BE
    return pl.pallas_call(
        _proj_body,
        grid=(nb,),
        in_specs=[
            pl.BlockSpec((_BE, D), lambda i: (i + s * nb, 0)),
            pl.BlockSpec((D, D), lambda i: (0, 0)),
            pl.BlockSpec((1, D), lambda i: (0, 0)),
        ],
        out_specs=pl.BlockSpec((_BE, D // 2), lambda i: (i, 0)),
        out_shape=jax.ShapeDtypeStruct((EPS, D // 2), jnp.int32),
    )


def _sc_body(s, x_hbm, e_hbm, src_hbm, dst_hbm, zeros_hbm, out_hbm,
             sidx, didx, xg_v, em_v, msg_v, agg_sh,
             gsem0, gsem1, esem0, esem1, ss0, ss1, ds0, ds1, cs0, cs1):
    cid = lax.axis_index("c")
    sid = lax.axis_index("s")
    wid = cid * NS + sid
    gsems = (gsem0, gsem1)
    esems = (esem0, esem1)
    ssems = (ss0, ss1)
    dsems = (ds0, ds1)
    csems = (cs0, cs1)

    pltpu.sync_copy(zeros_hbm.at[pl.ds(sid * RPT, RPT)],
                    agg_sh.at[pl.ds(sid * RPT, RPT)])
    plsc.subcore_barrier()

    ibase = s * EPS + wid * EPW
    ebase = wid * EPW

    def sidx_desc(ci, b):
        return pltpu.make_async_copy(src_hbm.at[pl.ds(ibase + ci * B, B)],
                                     sidx.at[b], ssems[b])

    def didx_desc(ci, b):
        return pltpu.make_async_copy(dst_hbm.at[pl.ds(ibase + ci * B, B)],
                                     didx.at[b], dsems[b])

    def scat_desc(b):
        return pltpu.make_async_copy(msg_v.at[b], agg_sh.at[didx.at[b]],
                                     csems[b])

    def gather_desc(ci, b):
        return (
            pltpu.make_async_copy(x_hbm.at[sidx.at[b]], xg_v.at[b], gsems[b]),
            pltpu.make_async_copy(e_hbm.at[pl.ds(ebase + ci * B, B)],
                                  em_v.at[b], esems[b]),
        )

    def start(descs):
        for d_ in descs:
            d_.start()

    def wait(descs):
        for d_ in descs:
            d_.wait()

    def chunk_step(ci, b):
        nb = 1 - b

        @pl.when(ci + 1 < C)
        def _():
            sidx_desc(ci + 1, nb).wait()
            start(gather_desc(ci + 1, nb))

        @pl.when(ci >= 2)
        def _():
            scat_desc(b).wait()

        wait(gather_desc(ci, b))

        @pl.when(ci + 2 < C)
        def _():
            sidx_desc(ci + 2, b).start()
            didx_desc(ci + 2, b).start()

        def rows(ri, c2):
            m = jnp.int32(-65536)
            for dr in range(4):
                r = 4 * ri + dr
                for k in range(D // (2 * L)):
                    ks = pl.ds(k * L, L)
                    hs = pl.ds(D // 2 + k * L, L)
                    ew = em_v[b, r, ks]
                    lo = lax.bitcast_convert_type(ew << 16, jnp.float32)
                    hi = lax.bitcast_convert_type(ew & m, jnp.float32)
                    msg_v[b, r, ks] = jnp.maximum(xg_v[b, r, ks] + lo, 0.0)
                    msg_v[b, r, hs] = jnp.maximum(xg_v[b, r, hs] + hi, 0.0)
            return c2

        lax.fori_loop(0, B // 4, rows, 0)
        didx_desc(ci, b).wait()
        pltpu.async_copy(msg_v.at[b], agg_sh.at[didx.at[b]], csems[b],
                         add=True)

    sidx_desc(0, 0).start()
    didx_desc(0, 0).start()
    sidx_desc(0, 0).wait()
    start(gather_desc(0, 0))
    sidx_desc(1, 1).start()
    didx_desc(1, 1).start()

    def pair(g, carry):
        chunk_step(2 * g, 0)
        chunk_step(2 * g + 1, 1)
        return carry

    lax.fori_loop(0, C // 2, pair, 0)
    if C % 2:
        chunk_step(C - 1, 0)
    scat_desc((C - 2) % 2).wait()
    scat_desc((C - 1) % 2).wait()
    plsc.subcore_barrier()

    pltpu.sync_copy(agg_sh.at[pl.ds(sid * RPT, RPT)],
                    out_hbm.at[cid, pl.ds(sid * RPT, RPT)])


def _make_sc(s):
    return pl.kernel(
        functools.partial(_sc_body, s),
        out_type=jax.ShapeDtypeStruct((NC, NP, D), jnp.float32),
        mesh=plsc.VectorSubcoreMesh(core_axis_name="c", subcore_axis_name="s"),
        scratch_types=[
            pltpu.VMEM((2, B), jnp.int32),
            pltpu.VMEM((2, B), jnp.int32),
            pltpu.VMEM((2, B, D), jnp.float32),
            pltpu.VMEM((2, B, D // 2), jnp.int32),
            pltpu.VMEM((2, B, D), jnp.float32),
            pltpu.VMEM_SHARED((NP, D), jnp.float32),
        ] + [pltpu.SemaphoreType.DMA] * 10,
    )


def _mlp_body(x_ref, p_ref, w1_ref, b1_ref, g_ref, bt_ref, w2_ref,
              b2_ref, eps_ref, o_ref):
    xv = x_ref[...]
    agg = p_ref[0, :N, :] + p_ref[1, :N, :]
    h = (1.0 + eps_ref[...]) * xv + agg
    h1 = jnp.dot(h, w1_ref[...], preferred_element_type=jnp.float32) + b1_ref[...]
    mean = jnp.mean(h1, axis=0, keepdims=True)
    ctr = h1 - mean
    var = jnp.mean(ctr * ctr, axis=0, keepdims=True)
    hn = ctr * lax.rsqrt(var + BN_EPS) * g_ref[...] + bt_ref[...]
    h2 = jnp.maximum(hn, 0.0)
    o_ref[...] = (
        xv + jnp.dot(h2, w2_ref[...], preferred_element_type=jnp.float32)
        + b2_ref[...]
    )


_mlp = pl.pallas_call(
    _mlp_body,
    out_shape=jax.ShapeDtypeStruct((N, D), jnp.float32),
)

_projs = [_make_proj(s) for s in range(S)]
_scs = [_make_sc(s) for s in range(S)]


def kernel(x, edge_index, edge_attr, We, be, W1, b1, gamma, beta, W2, b2, eps):
    src = edge_index[0]
    dst = edge_index[1]
    WeT = We.T
    be1 = be.reshape(1, D)
    zeros = jnp.zeros((NP, D), jnp.float32)
    e0 = _projs[0](edge_attr, WeT, be1)
    parts = _scs[0](x, e0, src, dst, zeros)
    out = _mlp(x, parts, W1.T, b1.reshape(1, D),
               gamma.reshape(1, D), beta.reshape(1, D), W2.T,
               b2.reshape(1, D), eps.reshape(1, 1))
    return (out, edge_attr)


# speedup vs baseline: 1.5418x; 1.0801x over previous
import functools

import jax
import jax.numpy as jnp
from jax import lax
from jax.experimental import pallas as pl
from jax.experimental.pallas import tpu as pltpu
from jax.experimental.pallas import tpu_sc as plsc

N = 10000
E = 320000
D = 128
BN_EPS = 1e-5

S = 1
EPS = E // S
NC, NS, L = 2, 16, 16
NW = NC * NS
EPW = EPS // NW
B = 40
C = EPW // B
NP = 10240
RPT = NP // NS


_BE = 6400


def _rne_bf16_bits(w):
    return w + jnp.int32(0x7FFF) + ((w >> 16) & jnp.int32(1))


def _proj_body(a_ref, w_ref, b_ref, o_ref):
    y = (
        jnp.dot(a_ref[...], w_ref[...], preferred_element_type=jnp.float32)
        + b_ref[...]
    )
    w1 = _rne_bf16_bits(lax.bitcast_convert_type(y[:, : D // 2], jnp.int32))
    w2 = _rne_bf16_bits(lax.bitcast_convert_type(y[:, D // 2 :], jnp.int32))
    o_ref[...] = ((w1 >> 16) & jnp.int32(0xFFFF)) | (w2 & jnp.int32(-65536))


def _make_proj(s):
    nb = EPS // _BE
    return pl.pallas_call(
        _proj_body,
        grid=(nb,),
        in_specs=[
            pl.BlockSpec((_BE, D), lambda i: (i + s * nb, 0)),
            pl.BlockSpec((D, D), lambda i: (0, 0)),
            pl.BlockSpec((1, D), lambda i: (0, 0)),
        ],
        out_specs=pl.BlockSpec((_BE, D // 2), lambda i: (i, 0)),
        out_shape=jax.ShapeDtypeStruct((EPS, D // 2), jnp.int32),
    )


def _sc_body(s, x_hbm, e_hbm, src_hbm, dst_hbm, out_hbm,
             sidx, didx, xg_v, em_v, msg_v, agg_sh,
             gsem0, gsem1, esem0, esem1, ss0, ss1, ds0, ds1, cs0, cs1):
    cid = lax.axis_index("c")
    sid = lax.axis_index("s")
    wid = cid * NS + sid
    gsems = (gsem0, gsem1)
    esems = (esem0, esem1)
    ssems = (ss0, ss1)
    dsems = (ds0, ds1)
    csems = (cs0, cs1)

    def zrow(r, c2):
        for k in range(D // L):
            msg_v[0, r, pl.ds(k * L, L)] = jnp.zeros((L,), jnp.float32)
        return c2

    lax.fori_loop(0, B, zrow, 0)
    def zcopy(i, c2):
        pltpu.sync_copy(msg_v.at[0],
                        agg_sh.at[pl.ds(sid * RPT + i * B, B)])
        return c2

    lax.fori_loop(0, RPT // B, zcopy, 0)
    plsc.subcore_barrier()

    ibase = s * EPS + wid * EPW
    ebase = wid * EPW

    def sidx_desc(ci, b):
        return pltpu.make_async_copy(src_hbm.at[pl.ds(ibase + ci * B, B)],
                                     sidx.at[b], ssems[b])

    def didx_desc(ci, b):
        return pltpu.make_async_copy(dst_hbm.at[pl.ds(ibase + ci * B, B)],
                                     didx.at[b], dsems[b])

    def scat_desc(b):
        return pltpu.make_async_copy(msg_v.at[b], agg_sh.at[didx.at[b]],
                                     csems[b])

    def gather_desc(ci, b):
        return (
            pltpu.make_async_copy(x_hbm.at[sidx.at[b]], xg_v.at[b], gsems[b]),
            pltpu.make_async_copy(e_hbm.at[pl.ds(ebase + ci * B, B)],
                                  em_v.at[b], esems[b]),
        )

    def start(descs):
        for d_ in descs:
            d_.start()

    def wait(descs):
        for d_ in descs:
            d_.wait()

    def chunk_step(ci, b):
        nb = 1 - b

        @pl.when(ci + 1 < C)
        def _():
            sidx_desc(ci + 1, nb).wait()
            start(gather_desc(ci + 1, nb))

        @pl.when(ci >= 2)
        def _():
            scat_desc(b).wait()

        wait(gather_desc(ci, b))

        @pl.when(ci + 2 < C)
        def _():
            sidx_desc(ci + 2, b).start()
            didx_desc(ci + 2, b).start()

        def rows(ri, c2):
            m = jnp.int32(-65536)
            for dr in range(4):
                r = 4 * ri + dr
                for k in range(D // (2 * L)):
                    ks = pl.ds(k * L, L)
                    hs = pl.ds(D // 2 + k * L, L)
                    ew = em_v[b, r, ks]
                    lo = lax.bitcast_convert_type(ew << 16, jnp.float32)
                    hi = lax.bitcast_convert_type(ew & m, jnp.float32)
                    msg_v[b, r, ks] = jnp.maximum(xg_v[b, r, ks] + lo, 0.0)
                    msg_v[b, r, hs] = jnp.maximum(xg_v[b, r, hs] + hi, 0.0)
            return c2

        lax.fori_loop(0, B // 4, rows, 0)
        didx_desc(ci, b).wait()
        pltpu.async_copy(msg_v.at[b], agg_sh.at[didx.at[b]], csems[b],
                         add=True)

    sidx_desc(0, 0).start()
    didx_desc(0, 0).start()
    sidx_desc(0, 0).wait()
    start(gather_desc(0, 0))
    sidx_desc(1, 1).start()
    didx_desc(1, 1).start()

    def pair(g, carry):
        chunk_step(2 * g, 0)
        chunk_step(2 * g + 1, 1)
        return carry

    lax.fori_loop(0, C // 2, pair, 0)
    if C % 2:
        chunk_step(C - 1, 0)
    scat_desc((C - 2) % 2).wait()
    scat_desc((C - 1) % 2).wait()
    plsc.subcore_barrier()

    pltpu.sync_copy(agg_sh.at[pl.ds(sid * RPT, RPT)],
                    out_hbm.at[cid, pl.ds(sid * RPT, RPT)])


def _make_sc(s):
    return pl.kernel(
        functools.partial(_sc_body, s),
        out_type=jax.ShapeDtypeStruct((NC, NP, D), jnp.float32),
        mesh=plsc.VectorSubcoreMesh(core_axis_name="c", subcore_axis_name="s"),
        scratch_types=[
            pltpu.VMEM((2, B), jnp.int32),
            pltpu.VMEM((2, B), jnp.int32),
            pltpu.VMEM((2, B, D), jnp.float32),
            pltpu.VMEM((2, B, D // 2), jnp.int32),
            pltpu.VMEM((2, B, D), jnp.float32),
            pltpu.VMEM_SHARED((NP, D), jnp.float32),
        ] + [pltpu.SemaphoreType.DMA] * 10,
    )


def _mlp_body(x_ref, p_ref, w1_ref, b1_ref, g_ref, bt_ref, w2_ref,
              b2_ref, eps_ref, o_ref):
    xv = x_ref[...]
    agg = p_ref[0, :N, :] + p_ref[1, :N, :]
    h = (1.0 + eps_ref[...]) * xv + agg
    h1 = jnp.dot(h, w1_ref[...], preferred_element_type=jnp.float32) + b1_ref[...]
    mean = jnp.mean(h1, axis=0, keepdims=True)
    ctr = h1 - mean
    var = jnp.mean(ctr * ctr, axis=0, keepdims=True)
    hn = ctr * lax.rsqrt(var + BN_EPS) * g_ref[...] + bt_ref[...]
    h2 = jnp.maximum(hn, 0.0)
    o_ref[...] = (
        xv + jnp.dot(h2, w2_ref[...], preferred_element_type=jnp.float32)
        + b2_ref[...]
    )


_mlp = pl.pallas_call(
    _mlp_body,
    out_shape=jax.ShapeDtypeStruct((N, D), jnp.float32),
)

_projs = [_make_proj(s) for s in range(S)]
_scs = [_make_sc(s) for s in range(S)]


def kernel(x, edge_index, edge_attr, We, be, W1, b1, gamma, beta, W2, b2, eps):
    src = edge_index[0]
    dst = edge_index[1]
    WeT = We.T
    be1 = be.reshape(1, D)
    e0 = _projs[0](edge_attr, WeT, be1)
    parts = _scs[0](x, e0, src, dst)
    out = _mlp(x, parts, W1.T, b1.reshape(1, D),
               gamma.reshape(1, D), beta.reshape(1, D), W2.T,
               b2.reshape(1, D), eps.reshape(1, 1))
    return (out, edge_attr)
